# baseline (speedup 1.0000x reference)
import functools

import jax
import jax.numpy as jnp
from jax import lax
from jax.experimental import pallas as pl
from jax.experimental.pallas import tpu as pltpu
from jax.experimental.pallas import tpu_sc as plsc

N_HID = 128
LANES = 16
NC = 2
NS = 16
NW = NC * NS
CHUNK = 208
GSUB = 104


def _project_table(emb, W, b2):
    M = emb.shape[0]
    BM = 27760
    grid = (pl.cdiv(M, BM),)

    def body(e_ref, w_ref, b_ref, o_ref):
        acc = jax.lax.dot_general(
            e_ref[...], w_ref[...],
            dimension_numbers=(((1,), (1,)), ((), ())),
            preferred_element_type=jnp.float32,
        )
        o_ref[...] = acc + b_ref[0, :][None, :]

    return pl.pallas_call(
        body,
        grid=grid,
        in_specs=[
            pl.BlockSpec((BM, N_HID), lambda i: (i, 0)),
            pl.BlockSpec((N_HID, N_HID), lambda i: (0, 0)),
            pl.BlockSpec((8, N_HID), lambda i: (0, 0)),
        ],
        out_specs=pl.BlockSpec((BM, N_HID), lambda i: (i, 0)),
        out_shape=jax.ShapeDtypeStruct((M, N_HID), jnp.float32),
    )(emb, W, b2)


def _gather_add(P, t, x):
    N = t.shape[0]
    assert N % NW == 0
    rows_w = N // NW
    nfull = rows_w // CHUNK
    rem = rows_w - nfull * CHUNK
    assert rem % 8 == 0 and rem < CHUNK and nfull >= 4 and nfull % 2 == 0

    mesh = plsc.VectorSubcoreMesh(core_axis_name="c", subcore_axis_name="s")

    @functools.partial(
        pl.kernel,
        mesh=mesh,
        out_type=jax.ShapeDtypeStruct((N, N_HID), jnp.float32),
        compiler_params=pltpu.CompilerParams(
            needs_layout_passes=False, use_tc_tiling_on_sc=False),
        scratch_types=[
            pltpu.VMEM((rows_w,), jnp.int32),
            pltpu.VMEM((2, CHUNK, N_HID), jnp.float32),
            pltpu.VMEM((2, CHUNK, N_HID), jnp.float32),
            pltpu.SemaphoreType.DMA,
            pltpu.SemaphoreType.DMA,
            pltpu.SemaphoreType.DMA,
            pltpu.SemaphoreType.DMA,
            pltpu.SemaphoreType.DMA,
            pltpu.SemaphoreType.DMA,
        ],
    )
    def k(p_hbm, t_hbm, x_hbm, out_hbm, t_v, rows_v, x_v,
          g0, g1, xs0, xs1, os0, os1):
        wid = lax.axis_index("s") * NC + lax.axis_index("c")
        wbase = wid * rows_w
        gsem = (g0, g1)
        xsem = (xs0, xs1)
        osem = (os0, os1)

        pltpu.sync_copy(t_hbm.at[pl.ds(wbase, rows_w)], t_v)

        def gather_descs(c, s):
            return [
                pltpu.make_async_copy(
                    p_hbm.at[t_v.at[pl.ds(c * CHUNK + u * GSUB, GSUB)]],
                    rows_v.at[s, pl.ds(u * GSUB, GSUB)], gsem[s])
                for u in range(CHUNK // GSUB)
            ]

        def x_desc(c, s):
            return pltpu.make_async_copy(
                x_hbm.at[pl.ds(wbase + c * CHUNK, CHUNK)], x_v.at[s], xsem[s])

        def o_desc(c, s):
            return pltpu.make_async_copy(
                x_v.at[s], out_hbm.at[pl.ds(wbase + c * CHUNK, CHUNK)],
                osem[s])

        def issue(c, s):
            for d in gather_descs(c, s):
                d.start()
            x_desc(c, s).start()

        def unpack_add_row(s, r):
            for j in range(N_HID // LANES):
                sl = pl.ds(j * LANES, LANES)
                x_v[s, r, sl] = x_v[s, r, sl] + rows_v[s, r, sl]

        def add_chunk(s, n):
            def add_row(r, _):
                unpack_add_row(s, r)
                return 0
            lax.fori_loop(0, n, add_row, 0)

        def process(c, s, prefetch, wait_store):
            for d in gather_descs(c, s):
                d.wait()
            x_desc(c, s).wait()
            if prefetch:
                if wait_store:
                    o_desc(c - 1, 1 - s).wait()
                issue(c + 1, 1 - s)
            add_chunk(s, CHUNK)
            o_desc(c, s).start()

        issue(0, 0)
        process(0, 0, True, False)
        process(1, 1, True, True)

        def pair_body(i, _):
            process(2 * i, 0, True, True)
            process(2 * i + 1, 1, True, True)
            return 0
        lax.fori_loop(1, nfull // 2 - 1, pair_body, 0)

        process(nfull - 2, 0, True, True)
        process(nfull - 1, 1, False, False)

        o_desc(nfull - 2, 0).wait()

        if rem:
            rbase = wbase + nfull * CHUNK
            pltpu.make_async_copy(
                p_hbm.at[t_v.at[pl.ds(nfull * CHUNK, rem)]],
                rows_v.at[0, pl.ds(0, rem)], gsem[0]).start()
            pltpu.sync_copy(x_hbm.at[pl.ds(rbase, rem)],
                            x_v.at[0, pl.ds(0, rem)])
            pltpu.make_async_copy(
                p_hbm.at[t_v.at[pl.ds(nfull * CHUNK, rem)]],
                rows_v.at[0, pl.ds(0, rem)], gsem[0]).wait()
            add_chunk(0, rem)
            pltpu.sync_copy(x_v.at[0, pl.ds(0, rem)],
                            out_hbm.at[pl.ds(rbase, rem)])

        o_desc(nfull - 1, 1).wait()

    return k(P, t, x)


def kernel(x, t, emb, W, b):
    t = t.astype(jnp.int32)
    b2 = jnp.broadcast_to(b[None, :], (8, N_HID))
    P = _project_table(emb, W, b2)
    return _gather_add(P, t, x)


# speedup vs baseline: 1.0096x; 1.0096x over previous
import functools

import jax
import jax.numpy as jnp
from jax import lax
from jax.experimental import pallas as pl
from jax.experimental.pallas import tpu as pltpu
from jax.experimental.pallas import tpu_sc as plsc

N_HID = 128
LANES = 16
NC = 2
NS = 16
NW = NC * NS
CHUNK = 208
GSUB = 104


def _project_table(emb, W, b2):
    M = emb.shape[0]
    BM = 13880
    grid = (pl.cdiv(M, BM),)

    def body(e_ref, w_ref, b_ref, o_ref):
        acc = jax.lax.dot_general(
            e_ref[...], w_ref[...],
            dimension_numbers=(((1,), (1,)), ((), ())),
            preferred_element_type=jnp.float32,
        )
        o_ref[...] = acc + b_ref[0, :][None, :]

    return pl.pallas_call(
        body,
        grid=grid,
        in_specs=[
            pl.BlockSpec((BM, N_HID), lambda i: (i, 0)),
            pl.BlockSpec((N_HID, N_HID), lambda i: (0, 0)),
            pl.BlockSpec((8, N_HID), lambda i: (0, 0)),
        ],
        out_specs=pl.BlockSpec((BM, N_HID), lambda i: (i, 0)),
        out_shape=jax.ShapeDtypeStruct((M, N_HID), jnp.float32),
    )(emb, W, b2)


def _gather_add(P, t, x):
    N = t.shape[0]
    assert N % NW == 0
    rows_w = N // NW
    nfull = rows_w // CHUNK
    rem = rows_w - nfull * CHUNK
    assert rem % 8 == 0 and rem < CHUNK and nfull >= 4 and nfull % 2 == 0

    mesh = plsc.VectorSubcoreMesh(core_axis_name="c", subcore_axis_name="s")

    @functools.partial(
        pl.kernel,
        mesh=mesh,
        out_type=jax.ShapeDtypeStruct((N, N_HID), jnp.float32),
        compiler_params=pltpu.CompilerParams(
            needs_layout_passes=False, use_tc_tiling_on_sc=False),
        scratch_types=[
            pltpu.VMEM((rows_w,), jnp.int32),
            pltpu.VMEM((2, CHUNK, N_HID), jnp.float32),
            pltpu.VMEM((2, CHUNK, N_HID), jnp.float32),
            pltpu.SemaphoreType.DMA,
            pltpu.SemaphoreType.DMA,
            pltpu.SemaphoreType.DMA,
            pltpu.SemaphoreType.DMA,
            pltpu.SemaphoreType.DMA,
            pltpu.SemaphoreType.DMA,
        ],
    )
    def k(p_hbm, t_hbm, x_hbm, out_hbm, t_v, rows_v, x_v,
          g0, g1, xs0, xs1, os0, os1):
        wid = lax.axis_index("s") * NC + lax.axis_index("c")
        wbase = wid * rows_w
        gsem = (g0, g1)
        xsem = (xs0, xs1)
        osem = (os0, os1)

        pltpu.sync_copy(t_hbm.at[pl.ds(wbase, rows_w)], t_v)

        def gather_descs(c, s):
            return [
                pltpu.make_async_copy(
                    p_hbm.at[t_v.at[pl.ds(c * CHUNK + u * GSUB, GSUB)]],
                    rows_v.at[s, pl.ds(u * GSUB, GSUB)], gsem[s])
                for u in range(CHUNK // GSUB)
            ]

        def x_desc(c, s):
            return pltpu.make_async_copy(
                x_hbm.at[pl.ds(wbase + c * CHUNK, CHUNK)], x_v.at[s], xsem[s])

        def o_desc(c, s):
            return pltpu.make_async_copy(
                x_v.at[s], out_hbm.at[pl.ds(wbase + c * CHUNK, CHUNK)],
                osem[s])

        def issue(c, s):
            for d in gather_descs(c, s):
                d.start()
            x_desc(c, s).start()

        def unpack_add_row(s, r):
            for j in range(N_HID // LANES):
                sl = pl.ds(j * LANES, LANES)
                x_v[s, r, sl] = x_v[s, r, sl] + rows_v[s, r, sl]

        def add_chunk(s, n):
            def add_row(r, _):
                unpack_add_row(s, r)
                return 0
            lax.fori_loop(0, n, add_row, 0)

        def process(c, s, prefetch, wait_store):
            for d in gather_descs(c, s):
                d.wait()
            x_desc(c, s).wait()
            if prefetch:
                if wait_store:
                    o_desc(c - 1, 1 - s).wait()
                issue(c + 1, 1 - s)
            add_chunk(s, CHUNK)
            o_desc(c, s).start()

        issue(0, 0)
        process(0, 0, True, False)
        process(1, 1, True, True)

        def pair_body(i, _):
            process(2 * i, 0, True, True)
            process(2 * i + 1, 1, True, True)
            return 0
        lax.fori_loop(1, nfull // 2 - 1, pair_body, 0)

        process(nfull - 2, 0, True, True)
        process(nfull - 1, 1, False, False)

        o_desc(nfull - 2, 0).wait()

        if rem:
            rbase = wbase + nfull * CHUNK
            pltpu.make_async_copy(
                p_hbm.at[t_v.at[pl.ds(nfull * CHUNK, rem)]],
                rows_v.at[0, pl.ds(0, rem)], gsem[0]).start()
            pltpu.sync_copy(x_hbm.at[pl.ds(rbase, rem)],
                            x_v.at[0, pl.ds(0, rem)])
            pltpu.make_async_copy(
                p_hbm.at[t_v.at[pl.ds(nfull * CHUNK, rem)]],
                rows_v.at[0, pl.ds(0, rem)], gsem[0]).wait()
            add_chunk(0, rem)
            pltpu.sync_copy(x_v.at[0, pl.ds(0, rem)],
                            out_hbm.at[pl.ds(rbase, rem)])

        o_desc(nfull - 1, 1).wait()

    return k(P, t, x)


def kernel(x, t, emb, W, b):
    t = t.astype(jnp.int32)
    b2 = jnp.broadcast_to(b[None, :], (8, N_HID))
    P = _project_table(emb, W, b2)
    return _gather_add(P, t, x)
